# baseline (speedup 1.0000x reference)
import jax
import jax.numpy as jnp
from jax import lax
from jax.experimental import pallas as pl

_B, _C, _H, _K = 8, 768, 14, 5
_D = _H * _H
_BIG = 3e38
_FILL = 1e19
_OFFS = (-6, -5, -4, -3, -2, -1, 1, 2, 3, 4, 5, 6)


def _shift_col(col, o, fill):
    if o > 0:
        pad = jnp.full((o, 1), fill, col.dtype)
        return jnp.concatenate([col[o:, :], pad], axis=0)
    pad = jnp.full((-o, 1), fill, col.dtype)
    return jnp.concatenate([pad, col[:o, :]], axis=0)


def _to_bytes(col):
    y = lax.bitcast_convert_type(col, jnp.int32)
    planes = [
        (lax.shift_right_logical(y, 8 * i) & 255).astype(jnp.float32)
        for i in range(4)
    ]
    return jnp.concatenate(planes, axis=1)


def _from_bytes(rows):
    y = rows[0:1, :].astype(jnp.int32)
    for i in range(1, 4):
        y = y | lax.shift_left(rows[i:i + 1, :].astype(jnp.int32), 8 * i)
    return lax.bitcast_convert_type(y, jnp.float32)


def _regraph_body(x_ref, w_ref, b_ref, o_ref):
    x = x_ref[0]
    gap = jnp.sum(x, axis=1, keepdims=True) * (1.0 / _D)
    gap_t = jnp.transpose(gap)

    rid = lax.broadcasted_iota(jnp.int32, (_C, _C), 0)
    cid = lax.broadcasted_iota(jnp.int32, (_C, _C), 1)
    lt = (gap < gap_t) | ((gap == gap_t) & (rid < cid))
    rank_t = jnp.sum(jnp.where(lt, 1, 0), axis=0, keepdims=True)
    r_mat = jnp.where(rid == rank_t, jnp.float32(1.0), 0.0)

    p2v = jnp.transpose(_from_bytes(jnp.transpose(
        jnp.dot(r_mat, _to_bytes(gap),
                preferred_element_type=jnp.float32))))

    dd = jnp.concatenate(
        [(p2v - _shift_col(p2v, o, _FILL)) ** 2 for o in _OFFS], axis=1)
    d5 = None
    for _ in range(_K):
        d5 = jnp.min(dd, axis=1, keepdims=True)
        dd = jnp.where(dd <= d5, _BIG, dd)
    d6 = jnp.min(dd, axis=1, keepdims=True)
    thr = 0.5 * d5 + 0.5 * d6
    thr_u = _from_bytes(
        lax.dot_general(_to_bytes(thr), r_mat, (((0,), (0,)), ((), ())),
                        preferred_element_type=jnp.float32))

    diff = gap - gap_t
    d = jnp.where(rid == cid, _BIG, diff * diff)
    adj = (d <= thr_u) | (d <= jnp.transpose(thr_u))
    a_sym = jnp.where(adj, jnp.float32(1.0), 0.0)

    deg = jnp.sum(a_sym, axis=1, keepdims=True) + 1.0
    dinv = lax.rsqrt(deg)

    h = jnp.dot(x, w_ref[...], preferred_element_type=jnp.float32)
    hs = h * dinv
    agg = jnp.dot(a_sym, hs, preferred_element_type=jnp.float32) + hs
    out = jnp.maximum(agg * dinv + b_ref[...], 0.0) + x
    o_ref[0] = out


def kernel(feature_map, W, b, k):
    del k
    x = feature_map.reshape(_B, _C, _D)
    out = pl.pallas_call(
        _regraph_body,
        grid=(_B,),
        in_specs=[
            pl.BlockSpec((1, _C, _D), lambda i: (i, 0, 0)),
            pl.BlockSpec((_D, _D), lambda i: (0, 0)),
            pl.BlockSpec((1, _D), lambda i: (0, 0)),
        ],
        out_specs=pl.BlockSpec((1, _C, _D), lambda i: (i, 0, 0)),
        out_shape=jax.ShapeDtypeStruct((_B, _C, _D), jnp.float32),
    )(x, W, b.reshape(1, _D))
    return out.reshape(_B, _C, _H, _H)


# speedup vs baseline: 1.1025x; 1.1025x over previous
import jax
import jax.numpy as jnp
from jax import lax
from jax.experimental import pallas as pl

_B, _C, _H, _K = 8, 768, 14, 5
_D = _H * _H


def _regraph_body(x_ref, w_ref, b_ref, o_ref):
    x = x_ref[0]
    gap = jnp.sum(x, axis=1, keepdims=True) * (1.0 / _D)
    gap_t = jnp.transpose(gap)

    rid = lax.broadcasted_iota(jnp.int32, (_C, _C), 0)
    cid = lax.broadcasted_iota(jnp.int32, (_C, _C), 1)
    diff = gap - gap_t
    d = jnp.where(rid == cid, jnp.float32(3e38), diff * diff)

    a = jnp.zeros((_C, _C), jnp.float32)
    for _ in range(_K):
        m = jnp.min(d, axis=1, keepdims=True)
        ismin = d <= m
        first = jnp.min(jnp.where(ismin, cid, jnp.int32(2**30)),
                        axis=1, keepdims=True)
        sel = cid == first
        a = jnp.where(sel, jnp.float32(1.0), a)
        d = jnp.where(sel, jnp.float32(3e38), d)

    a_sym = jnp.maximum(a, jnp.transpose(a))
    deg = jnp.sum(a_sym, axis=1, keepdims=True) + 1.0
    dinv = lax.rsqrt(deg)

    h = jnp.dot(x, w_ref[...], preferred_element_type=jnp.float32)
    hs = h * dinv
    agg = jnp.dot(a_sym, hs, preferred_element_type=jnp.float32) + hs
    out = jnp.maximum(agg * dinv + b_ref[...], 0.0) + x
    o_ref[0] = out


def kernel(feature_map, W, b, k):
    del k
    x = feature_map.reshape(_B, _C, _D)
    out = pl.pallas_call(
        _regraph_body,
        grid=(_B,),
        in_specs=[
            pl.BlockSpec((1, _C, _D), lambda i: (i, 0, 0)),
            pl.BlockSpec((_D, _D), lambda i: (0, 0)),
            pl.BlockSpec((1, _D), lambda i: (0, 0)),
        ],
        out_specs=pl.BlockSpec((1, _C, _D), lambda i: (i, 0, 0)),
        out_shape=jax.ShapeDtypeStruct((_B, _C, _D), jnp.float32),
    )(x, W, b.reshape(1, _D))
    return out.reshape(_B, _C, _H, _H)
